# baseline (speedup 1.0000x reference)
import functools

import jax
import jax.numpy as jnp
from jax.experimental import pallas as pl
from jax.experimental.pallas import tpu as pltpu

B, T, H = 8, 1, 2048
E, K, I, SI = 64, 8, 512, 512
N = B * T
P = N * K
NBUF = 2


def _router_shared_kernel(h_ref, gate_w_ref, sh_gate_ref, sh_up_ref,
                          sh_down_ref, se_gate_ref,
                          sh_out_ref, act_ref, wmat_ref):
    hv = h_ref[:]

    logits = jax.lax.dot_general(
        hv, gate_w_ref[:], (((1,), (1,)), ((), ())),
        preferred_element_type=jnp.float32)
    col = jax.lax.broadcasted_iota(jnp.int32, (N, E), 1)
    masked = logits
    vals = []
    idxs = []
    neg_inf = jnp.float32(-jnp.inf)
    for k in range(K):
        m = jnp.max(masked, axis=1, keepdims=True)
        is_m = masked == m
        idx = jnp.min(jnp.where(is_m, col, E), axis=1, keepdims=True)
        vals.append(m)
        idxs.append(idx)
        masked = jnp.where(col == idx, neg_inf, masked)
    topv = jnp.concatenate(vals, axis=1)
    ex = jnp.exp(topv - topv[:, 0:1])
    wts = ex / jnp.sum(ex, axis=1, keepdims=True)

    wmat = jnp.zeros((N, E), jnp.float32)
    sel = jnp.zeros((N, E), jnp.int32)
    for k in range(K):
        hit = col == idxs[k]
        wmat = wmat + jnp.where(hit, wts[:, k:k + 1], jnp.float32(0.0))
        sel = sel | hit.astype(jnp.int32)
    wmat_ref[:] = wmat
    act_ref[:] = jnp.max(sel, axis=0, keepdims=True)

    g = jax.lax.dot_general(hv, sh_gate_ref[:], (((1,), (1,)), ((), ())),
                            preferred_element_type=jnp.float32)
    u = jax.lax.dot_general(hv, sh_up_ref[:], (((1,), (1,)), ((), ())),
                            preferred_element_type=jnp.float32)
    inter = g * jax.nn.sigmoid(g) * u
    so = jax.lax.dot_general(inter, sh_down_ref[:], (((1,), (1,)), ((), ())),
                             preferred_element_type=jnp.float32)
    se = jax.nn.sigmoid(
        jax.lax.dot_general(hv, se_gate_ref[:], (((1,), (1,)), ((), ())),
                            preferred_element_type=jnp.float32))
    sh_out_ref[:] = se * so


def _moe_ffn_kernel(act_ref,
                    h_ref, wmat_ref, sh_ref, gu_hbm, dn_hbm,
                    out_ref,
                    gu_buf, dn_buf, gu_sem, dn_sem):

    def start_fetch(e, slot):
        pltpu.make_async_copy(gu_hbm.at[e], gu_buf.at[slot],
                              gu_sem.at[slot]).start()
        pltpu.make_async_copy(dn_hbm.at[e], dn_buf.at[slot],
                              dn_sem.at[slot]).start()

    def wait_and_compute(e, slot):
        pltpu.make_async_copy(gu_hbm.at[e], gu_buf.at[slot],
                              gu_sem.at[slot]).wait()
        pltpu.make_async_copy(dn_hbm.at[e], dn_buf.at[slot],
                              dn_sem.at[slot]).wait()
        hv = h_ref[:]
        gup = jax.lax.dot_general(hv, gu_buf[slot], (((1,), (1,)), ((), ())),
                                  preferred_element_type=jnp.float32)
        gate = gup[:, :I]
        up = gup[:, I:]
        inter = gate * jax.nn.sigmoid(gate) * up
        eout = jax.lax.dot_general(inter, dn_buf[slot],
                                   (((1,), (1,)), ((), ())),
                                   preferred_element_type=jnp.float32)
        ecol = jax.lax.broadcasted_iota(jnp.int32, (N, E), 1)
        wcol = jnp.sum(jnp.where(ecol == e, wmat_ref[:], jnp.float32(0.0)),
                       axis=1, keepdims=True)
        out_ref[:] += wcol * eout

    def body(e, carry):
        cnt, prev_e = carry
        is_act = act_ref[e] == 1

        @pl.when(is_act)
        def _():
            start_fetch(e, jax.lax.rem(cnt, NBUF))

            @pl.when(cnt == 0)
            def _():
                out_ref[:] = sh_ref[:]

            @pl.when(cnt > 0)
            def _():
                wait_and_compute(prev_e, jax.lax.rem(cnt - 1, NBUF))

        cnt = jnp.where(is_act, cnt + 1, cnt)
        prev_e = jnp.where(is_act, e, prev_e)
        return cnt, prev_e

    cnt, prev_e = jax.lax.fori_loop(0, E, body, (jnp.int32(0), jnp.int32(0)))
    wait_and_compute(prev_e, jax.lax.rem(cnt - 1, NBUF))


@functools.partial(jax.jit, static_argnames=())
def _run(h, gate_w, experts_gate_up, experts_down, sh_gate_w, sh_up_w,
         sh_down_w, se_gate_w):
    h_flat = h.reshape(N, H)

    sh_out, act, wmat = pl.pallas_call(
        _router_shared_kernel,
        out_shape=(
            jax.ShapeDtypeStruct((N, H), jnp.float32),
            jax.ShapeDtypeStruct((1, E), jnp.int32),
            jax.ShapeDtypeStruct((N, E), jnp.float32),
        ),
    )(h_flat, gate_w, sh_gate_w, sh_up_w, sh_down_w, se_gate_w)

    grid_spec = pltpu.PrefetchScalarGridSpec(
        num_scalar_prefetch=1,
        grid=(1,),
        in_specs=[
            pl.BlockSpec((N, H), lambda i, a: (0, 0)),
            pl.BlockSpec((N, E), lambda i, a: (0, 0)),
            pl.BlockSpec((N, H), lambda i, a: (0, 0)),
            pl.BlockSpec(memory_space=pltpu.MemorySpace.HBM),
            pl.BlockSpec(memory_space=pltpu.MemorySpace.HBM),
        ],
        out_specs=pl.BlockSpec((N, H), lambda i, a: (0, 0)),
        scratch_shapes=[
            pltpu.VMEM((NBUF, 2 * I, H), jnp.float32),
            pltpu.VMEM((NBUF, H, I), jnp.float32),
            pltpu.SemaphoreType.DMA((NBUF,)),
            pltpu.SemaphoreType.DMA((NBUF,)),
        ],
    )
    out = pl.pallas_call(
        _moe_ffn_kernel,
        grid_spec=grid_spec,
        out_shape=jax.ShapeDtypeStruct((N, H), jnp.float32),
        compiler_params=pltpu.CompilerParams(
            dimension_semantics=("arbitrary",)),
    )(act.reshape(E), h_flat, wmat, sh_out, experts_gate_up, experts_down)

    return out.reshape(B, T, H)


def kernel(h, gate_w, experts_gate_up, experts_down, sh_gate_w, sh_up_w,
           sh_down_w, se_gate_w):
    return _run(h, gate_w, experts_gate_up, experts_down, sh_gate_w,
                sh_up_w, sh_down_w, se_gate_w)


# speedup vs baseline: 1.0111x; 1.0111x over previous
import functools

import jax
import jax.numpy as jnp
from jax.experimental import pallas as pl
from jax.experimental.pallas import tpu as pltpu

B, T, H = 8, 1, 2048
E, K, I, SI = 64, 8, 512, 512
N = B * T
NBUF = 2


def _moe_kernel(h_ref, gate_w_ref, sh_gate_ref, sh_up_ref, sh_down_ref,
                se_gate_ref, gu_hbm, dn_hbm,
                out_ref,
                gu_buf, dn_buf, gu_sem, dn_sem,
                act_vmem, act_smem, act_sem):
    hv = h_ref[:]

    logits = jax.lax.dot_general(
        hv, gate_w_ref[:], (((1,), (1,)), ((), ())),
        preferred_element_type=jnp.float32)
    col = jax.lax.broadcasted_iota(jnp.int32, (N, E), 1)
    masked = logits
    vals = []
    idxs = []
    neg_inf = jnp.float32(-jnp.inf)
    for k in range(K):
        m = jnp.max(masked, axis=1, keepdims=True)
        is_m = masked == m
        idx = jnp.min(jnp.where(is_m, col, E), axis=1, keepdims=True)
        vals.append(m)
        idxs.append(idx)
        masked = jnp.where(col == idx, neg_inf, masked)
    topv = jnp.concatenate(vals, axis=1)
    ex = jnp.exp(topv - topv[:, 0:1])
    wts = ex / jnp.sum(ex, axis=1, keepdims=True)

    wmat = jnp.zeros((N, E), jnp.float32)
    sel = jnp.zeros((N, E), jnp.int32)
    for k in range(K):
        hit = col == idxs[k]
        wmat = wmat + jnp.where(hit, wts[:, k:k + 1], jnp.float32(0.0))
        sel = sel | hit.astype(jnp.int32)
    act_vmem[:] = jnp.max(sel, axis=0, keepdims=True)

    cp = pltpu.make_async_copy(act_vmem, act_smem, act_sem)
    cp.start()
    cp.wait()

    def start_fetch(e, slot):
        pltpu.make_async_copy(gu_hbm.at[e], gu_buf.at[slot],
                              gu_sem.at[slot]).start()
        pltpu.make_async_copy(dn_hbm.at[e], dn_buf.at[slot],
                              dn_sem.at[slot]).start()

    def compute_shared():
        g = jax.lax.dot_general(hv, sh_gate_ref[:], (((1,), (1,)), ((), ())),
                                preferred_element_type=jnp.float32)
        u = jax.lax.dot_general(hv, sh_up_ref[:], (((1,), (1,)), ((), ())),
                                preferred_element_type=jnp.float32)
        inter = g * jax.nn.sigmoid(g) * u
        so = jax.lax.dot_general(inter, sh_down_ref[:],
                                 (((1,), (1,)), ((), ())),
                                 preferred_element_type=jnp.float32)
        se = jax.nn.sigmoid(
            jax.lax.dot_general(hv, se_gate_ref[:], (((1,), (1,)), ((), ())),
                                preferred_element_type=jnp.float32))
        out_ref[:] = se * so

    def wait_and_compute(e, slot):
        pltpu.make_async_copy(gu_hbm.at[e], gu_buf.at[slot],
                              gu_sem.at[slot]).wait()
        pltpu.make_async_copy(dn_hbm.at[e], dn_buf.at[slot],
                              dn_sem.at[slot]).wait()
        gup = jax.lax.dot_general(hv, gu_buf[slot], (((1,), (1,)), ((), ())),
                                  preferred_element_type=jnp.float32)
        gate = gup[:, :I]
        up = gup[:, I:]
        inter = gate * jax.nn.sigmoid(gate) * up
        eout = jax.lax.dot_general(inter, dn_buf[slot],
                                   (((1,), (1,)), ((), ())),
                                   preferred_element_type=jnp.float32)
        wcol = jnp.sum(jnp.where(col == e, wmat, jnp.float32(0.0)),
                       axis=1, keepdims=True)
        out_ref[:] += wcol * eout

    def body(e, carry):
        cnt, prev_e = carry
        is_act = act_smem[0, e] == 1

        @pl.when(is_act)
        def _():
            start_fetch(e, jax.lax.rem(cnt, NBUF))

            @pl.when(cnt == 0)
            def _():
                compute_shared()

            @pl.when(cnt > 0)
            def _():
                wait_and_compute(prev_e, jax.lax.rem(cnt - 1, NBUF))

        cnt = jnp.where(is_act, cnt + 1, cnt)
        prev_e = jnp.where(is_act, e, prev_e)
        return cnt, prev_e

    cnt, prev_e = jax.lax.fori_loop(0, E, body, (jnp.int32(0), jnp.int32(0)))
    wait_and_compute(prev_e, jax.lax.rem(cnt - 1, NBUF))


@functools.partial(jax.jit, static_argnames=())
def _run(h, gate_w, experts_gate_up, experts_down, sh_gate_w, sh_up_w,
         sh_down_w, se_gate_w):
    h_flat = h.reshape(N, H)

    out = pl.pallas_call(
        _moe_kernel,
        grid=(1,),
        in_specs=[
            pl.BlockSpec((N, H), lambda i: (0, 0)),
            pl.BlockSpec((E, H), lambda i: (0, 0)),
            pl.BlockSpec((SI, H), lambda i: (0, 0)),
            pl.BlockSpec((SI, H), lambda i: (0, 0)),
            pl.BlockSpec((H, SI), lambda i: (0, 0)),
            pl.BlockSpec((1, H), lambda i: (0, 0)),
            pl.BlockSpec(memory_space=pltpu.MemorySpace.HBM),
            pl.BlockSpec(memory_space=pltpu.MemorySpace.HBM),
        ],
        out_specs=pl.BlockSpec((N, H), lambda i: (0, 0)),
        out_shape=jax.ShapeDtypeStruct((N, H), jnp.float32),
        scratch_shapes=[
            pltpu.VMEM((NBUF, 2 * I, H), jnp.float32),
            pltpu.VMEM((NBUF, H, I), jnp.float32),
            pltpu.SemaphoreType.DMA((NBUF,)),
            pltpu.SemaphoreType.DMA((NBUF,)),
            pltpu.VMEM((1, E), jnp.int32),
            pltpu.SMEM((1, E), jnp.int32),
            pltpu.SemaphoreType.DMA,
        ],
        compiler_params=pltpu.CompilerParams(
            dimension_semantics=("arbitrary",)),
    )(h_flat, gate_w, sh_gate_w, sh_up_w, sh_down_w, se_gate_w,
      experts_gate_up, experts_down)

    return out.reshape(B, T, H)


def kernel(h, gate_w, experts_gate_up, experts_down, sh_gate_w, sh_up_w,
           sh_down_w, se_gate_w):
    return _run(h, gate_w, experts_gate_up, experts_down, sh_gate_w,
                sh_up_w, sh_down_w, se_gate_w)


# speedup vs baseline: 1.0273x; 1.0161x over previous
import functools

import jax
import jax.numpy as jnp
from jax.experimental import pallas as pl
from jax.experimental.pallas import tpu as pltpu

B, T, H = 8, 1, 2048
E, K, I, SI = 64, 8, 512, 512
N = B * T
NBUF = 2


def _moe_kernel(h_ref, gate_w_ref, sh_gate_hbm, sh_up_hbm, sh_down_hbm,
                se_gate_hbm, gu_hbm, dn_hbm,
                out_ref,
                gu_buf, dn_buf, gu_sem, dn_sem,
                act_vmem, act_smem, act_sem,
                shg_buf, shu_buf, shd_buf, seg_buf, sh_sem):
    shg_cp = pltpu.make_async_copy(sh_gate_hbm, shg_buf, sh_sem.at[0])
    shu_cp = pltpu.make_async_copy(sh_up_hbm, shu_buf, sh_sem.at[1])
    shd_cp = pltpu.make_async_copy(sh_down_hbm, shd_buf, sh_sem.at[2])
    seg_cp = pltpu.make_async_copy(se_gate_hbm, seg_buf, sh_sem.at[3])
    shg_cp.start()
    shu_cp.start()
    shd_cp.start()
    seg_cp.start()

    hv = h_ref[:]

    logits = jax.lax.dot_general(
        hv, gate_w_ref[:], (((1,), (1,)), ((), ())),
        preferred_element_type=jnp.float32)
    col = jax.lax.broadcasted_iota(jnp.int32, (N, E), 1)
    masked = logits
    vals = []
    idxs = []
    neg_inf = jnp.float32(-jnp.inf)
    for k in range(K):
        m = jnp.max(masked, axis=1, keepdims=True)
        is_m = masked == m
        idx = jnp.min(jnp.where(is_m, col, E), axis=1, keepdims=True)
        vals.append(m)
        idxs.append(idx)
        masked = jnp.where(col == idx, neg_inf, masked)
    topv = jnp.concatenate(vals, axis=1)
    ex = jnp.exp(topv - topv[:, 0:1])
    wts = ex / jnp.sum(ex, axis=1, keepdims=True)

    wmat = jnp.zeros((N, E), jnp.float32)
    sel = jnp.zeros((N, E), jnp.int32)
    for k in range(K):
        hit = col == idxs[k]
        wmat = wmat + jnp.where(hit, wts[:, k:k + 1], jnp.float32(0.0))
        sel = sel | hit.astype(jnp.int32)
    act_vmem[:] = jnp.max(sel, axis=0, keepdims=True)

    cp = pltpu.make_async_copy(act_vmem, act_smem, act_sem)
    cp.start()
    cp.wait()

    def start_fetch(e, slot):
        pltpu.make_async_copy(gu_hbm.at[e], gu_buf.at[slot],
                              gu_sem.at[slot]).start()
        pltpu.make_async_copy(dn_hbm.at[e], dn_buf.at[slot],
                              dn_sem.at[slot]).start()

    def compute_shared():
        shg_cp.wait()
        shu_cp.wait()
        shd_cp.wait()
        seg_cp.wait()
        g = jax.lax.dot_general(hv, shg_buf[:], (((1,), (1,)), ((), ())),
                                preferred_element_type=jnp.float32)
        u = jax.lax.dot_general(hv, shu_buf[:], (((1,), (1,)), ((), ())),
                                preferred_element_type=jnp.float32)
        inter = g * jax.nn.sigmoid(g) * u
        so = jax.lax.dot_general(inter, shd_buf[:],
                                 (((1,), (1,)), ((), ())),
                                 preferred_element_type=jnp.float32)
        se = jax.nn.sigmoid(
            jax.lax.dot_general(hv, seg_buf[:], (((1,), (1,)), ((), ())),
                                preferred_element_type=jnp.float32))
        out_ref[:] = se * so

    def wait_and_compute(e, slot):
        pltpu.make_async_copy(gu_hbm.at[e], gu_buf.at[slot],
                              gu_sem.at[slot]).wait()
        pltpu.make_async_copy(dn_hbm.at[e], dn_buf.at[slot],
                              dn_sem.at[slot]).wait()
        gup = jax.lax.dot_general(hv, gu_buf[slot], (((1,), (1,)), ((), ())),
                                  preferred_element_type=jnp.float32)
        gate = gup[:, :I]
        up = gup[:, I:]
        inter = gate * jax.nn.sigmoid(gate) * up
        eout = jax.lax.dot_general(inter, dn_buf[slot],
                                   (((1,), (1,)), ((), ())),
                                   preferred_element_type=jnp.float32)
        wcol = jnp.sum(jnp.where(col == e, wmat, jnp.float32(0.0)),
                       axis=1, keepdims=True)
        out_ref[:] += wcol * eout

    def body(e, carry):
        cnt, prev_e = carry
        is_act = act_smem[0, e] == 1

        @pl.when(is_act)
        def _():
            start_fetch(e, jax.lax.rem(cnt, NBUF))

            @pl.when(cnt == 0)
            def _():
                compute_shared()

            @pl.when(cnt > 0)
            def _():
                wait_and_compute(prev_e, jax.lax.rem(cnt - 1, NBUF))

        cnt = jnp.where(is_act, cnt + 1, cnt)
        prev_e = jnp.where(is_act, e, prev_e)
        return cnt, prev_e

    cnt, prev_e = jax.lax.fori_loop(0, E, body, (jnp.int32(0), jnp.int32(0)))
    wait_and_compute(prev_e, jax.lax.rem(cnt - 1, NBUF))


@functools.partial(jax.jit, static_argnames=())
def _run(h, gate_w, experts_gate_up, experts_down, sh_gate_w, sh_up_w,
         sh_down_w, se_gate_w):
    h_flat = h.reshape(N, H)

    out = pl.pallas_call(
        _moe_kernel,
        grid=(1,),
        in_specs=[
            pl.BlockSpec((N, H), lambda i: (0, 0)),
            pl.BlockSpec((E, H), lambda i: (0, 0)),
            pl.BlockSpec(memory_space=pltpu.MemorySpace.HBM),
            pl.BlockSpec(memory_space=pltpu.MemorySpace.HBM),
            pl.BlockSpec(memory_space=pltpu.MemorySpace.HBM),
            pl.BlockSpec(memory_space=pltpu.MemorySpace.HBM),
            pl.BlockSpec(memory_space=pltpu.MemorySpace.HBM),
            pl.BlockSpec(memory_space=pltpu.MemorySpace.HBM),
        ],
        out_specs=pl.BlockSpec((N, H), lambda i: (0, 0)),
        out_shape=jax.ShapeDtypeStruct((N, H), jnp.float32),
        scratch_shapes=[
            pltpu.VMEM((NBUF, 2 * I, H), jnp.float32),
            pltpu.VMEM((NBUF, H, I), jnp.float32),
            pltpu.SemaphoreType.DMA((NBUF,)),
            pltpu.SemaphoreType.DMA((NBUF,)),
            pltpu.VMEM((1, E), jnp.int32),
            pltpu.SMEM((1, E), jnp.int32),
            pltpu.SemaphoreType.DMA,
            pltpu.VMEM((SI, H), jnp.float32),
            pltpu.VMEM((SI, H), jnp.float32),
            pltpu.VMEM((H, SI), jnp.float32),
            pltpu.VMEM((1, H), jnp.float32),
            pltpu.SemaphoreType.DMA((4,)),
        ],
        compiler_params=pltpu.CompilerParams(
            dimension_semantics=("arbitrary",)),
    )(h_flat, gate_w, sh_gate_w, sh_up_w, sh_down_w, se_gate_w,
      experts_gate_up, experts_down)

    return out.reshape(B, T, H)


def kernel(h, gate_w, experts_gate_up, experts_down, sh_gate_w, sh_up_w,
           sh_down_w, se_gate_w):
    return _run(h, gate_w, experts_gate_up, experts_down, sh_gate_w,
                sh_up_w, sh_down_w, se_gate_w)


# speedup vs baseline: 1.0275x; 1.0002x over previous
import functools

import jax
import jax.numpy as jnp
from jax.experimental import pallas as pl
from jax.experimental.pallas import tpu as pltpu

B, T, H = 8, 1, 2048
E, K, I, SI = 64, 8, 512, 512
N = B * T
NBUF = 2


def _moe_kernel(h_ref, gate_w_ref, sh_gate_hbm, sh_up_hbm, sh_down_hbm,
                se_gate_hbm, gu_hbm, dn_hbm,
                out_ref,
                gu_buf, dn_buf, gu_sem, dn_sem,
                act_vmem, act_smem, act_sem,
                shg_buf, shu_buf, shd_buf, seg_buf, sh_sem):
    shg_cp = pltpu.make_async_copy(sh_gate_hbm, shg_buf, sh_sem.at[0])
    shu_cp = pltpu.make_async_copy(sh_up_hbm, shu_buf, sh_sem.at[1])
    shd_cp = pltpu.make_async_copy(sh_down_hbm, shd_buf, sh_sem.at[2])
    seg_cp = pltpu.make_async_copy(se_gate_hbm, seg_buf, sh_sem.at[3])
    shg_cp.start()
    shu_cp.start()
    shd_cp.start()
    seg_cp.start()

    hv = h_ref[:]

    logits = jax.lax.dot_general(
        hv, gate_w_ref[:], (((1,), (1,)), ((), ())),
        preferred_element_type=jnp.float32)
    col = jax.lax.broadcasted_iota(jnp.int32, (N, E), 1)
    masked = logits
    vals = []
    idxs = []
    neg_inf = jnp.float32(-jnp.inf)
    for k in range(K):
        m = jnp.max(masked, axis=1, keepdims=True)
        is_m = masked == m
        idx = jnp.min(jnp.where(is_m, col, E), axis=1, keepdims=True)
        vals.append(m)
        idxs.append(idx)
        masked = jnp.where(col == idx, neg_inf, masked)
    topv = jnp.concatenate(vals, axis=1)
    ex = jnp.exp(topv - topv[:, 0:1])
    wts = ex / jnp.sum(ex, axis=1, keepdims=True)

    wmat = jnp.zeros((N, E), jnp.float32)
    sel = jnp.zeros((N, E), jnp.int32)
    for k in range(K):
        hit = col == idxs[k]
        wmat = wmat + jnp.where(hit, wts[:, k:k + 1], jnp.float32(0.0))
        sel = sel | hit.astype(jnp.int32)
    act_vmem[:] = jnp.max(sel, axis=0, keepdims=True)

    cp = pltpu.make_async_copy(act_vmem, act_smem, act_sem)
    cp.start()
    cp.wait()

    def start_fetch(e, slot):
        pltpu.make_async_copy(gu_hbm.at[e], gu_buf.at[slot],
                              gu_sem.at[slot]).start()
        pltpu.make_async_copy(dn_hbm.at[e], dn_buf.at[slot],
                              dn_sem.at[slot]).start()

    def compute_shared():
        shg_cp.wait()
        shu_cp.wait()
        shd_cp.wait()
        seg_cp.wait()
        g = jax.lax.dot_general(hv, shg_buf[:], (((1,), (1,)), ((), ())),
                                preferred_element_type=jnp.float32)
        u = jax.lax.dot_general(hv, shu_buf[:], (((1,), (1,)), ((), ())),
                                preferred_element_type=jnp.float32)
        inter = g * jax.nn.sigmoid(g) * u
        so = jax.lax.dot_general(inter, shd_buf[:],
                                 (((1,), (1,)), ((), ())),
                                 preferred_element_type=jnp.float32)
        se = jax.nn.sigmoid(
            jax.lax.dot_general(hv, seg_buf[:], (((1,), (1,)), ((), ())),
                                preferred_element_type=jnp.float32))
        out_ref[:] = se * so

    def wait_and_compute(e, slot):
        pltpu.make_async_copy(gu_hbm.at[e], gu_buf.at[slot],
                              gu_sem.at[slot]).wait()
        pltpu.make_async_copy(dn_hbm.at[e], dn_buf.at[slot],
                              dn_sem.at[slot]).wait()
        gup = jax.lax.dot_general(hv, gu_buf[slot], (((1,), (1,)), ((), ())),
                                  preferred_element_type=jnp.float32)
        gate = gup[:, :I]
        up = gup[:, I:]
        inter = gate * jax.nn.sigmoid(gate) * up
        eout = jax.lax.dot_general(inter, dn_buf[slot],
                                   (((1,), (1,)), ((), ())),
                                   preferred_element_type=jnp.float32)
        wcol = jnp.sum(jnp.where(col == e, wmat, jnp.float32(0.0)),
                       axis=1, keepdims=True)
        out_ref[:] += wcol * eout

    def body(e, carry):
        cnt, prev_e = carry
        is_act = act_smem[0, e] == 1

        @pl.when(is_act)
        def _():
            start_fetch(e, jax.lax.rem(cnt, NBUF))

            @pl.when(cnt == 0)
            def _():
                compute_shared()

            @pl.when(cnt > 0)
            def _():
                wait_and_compute(prev_e, jax.lax.rem(cnt - 1, NBUF))

        cnt = jnp.where(is_act, cnt + 1, cnt)
        prev_e = jnp.where(is_act, e, prev_e)
        return cnt, prev_e

    cnt, prev_e = jax.lax.fori_loop(0, E, body, (jnp.int32(0), jnp.int32(0)),
                                    unroll=4)
    wait_and_compute(prev_e, jax.lax.rem(cnt - 1, NBUF))


@functools.partial(jax.jit, static_argnames=())
def _run(h, gate_w, experts_gate_up, experts_down, sh_gate_w, sh_up_w,
         sh_down_w, se_gate_w):
    h_flat = h.reshape(N, H)

    out = pl.pallas_call(
        _moe_kernel,
        grid=(1,),
        in_specs=[
            pl.BlockSpec((N, H), lambda i: (0, 0)),
            pl.BlockSpec((E, H), lambda i: (0, 0)),
            pl.BlockSpec(memory_space=pltpu.MemorySpace.HBM),
            pl.BlockSpec(memory_space=pltpu.MemorySpace.HBM),
            pl.BlockSpec(memory_space=pltpu.MemorySpace.HBM),
            pl.BlockSpec(memory_space=pltpu.MemorySpace.HBM),
            pl.BlockSpec(memory_space=pltpu.MemorySpace.HBM),
            pl.BlockSpec(memory_space=pltpu.MemorySpace.HBM),
        ],
        out_specs=pl.BlockSpec((N, H), lambda i: (0, 0)),
        out_shape=jax.ShapeDtypeStruct((N, H), jnp.float32),
        scratch_shapes=[
            pltpu.VMEM((NBUF, 2 * I, H), jnp.float32),
            pltpu.VMEM((NBUF, H, I), jnp.float32),
            pltpu.SemaphoreType.DMA((NBUF,)),
            pltpu.SemaphoreType.DMA((NBUF,)),
            pltpu.VMEM((1, E), jnp.int32),
            pltpu.SMEM((1, E), jnp.int32),
            pltpu.SemaphoreType.DMA,
            pltpu.VMEM((SI, H), jnp.float32),
            pltpu.VMEM((SI, H), jnp.float32),
            pltpu.VMEM((H, SI), jnp.float32),
            pltpu.VMEM((1, H), jnp.float32),
            pltpu.SemaphoreType.DMA((4,)),
        ],
        compiler_params=pltpu.CompilerParams(
            dimension_semantics=("arbitrary",)),
    )(h_flat, gate_w, sh_gate_w, sh_up_w, sh_down_w, se_gate_w,
      experts_gate_up, experts_down)

    return out.reshape(B, T, H)


def kernel(h, gate_w, experts_gate_up, experts_down, sh_gate_w, sh_up_w,
           sh_down_w, se_gate_w):
    return _run(h, gate_w, experts_gate_up, experts_down, sh_gate_w,
                sh_up_w, sh_down_w, se_gate_w)
